# Initial kernel scaffold; baseline (speedup 1.0000x reference)
import jax
import jax.numpy as jnp
from jax.experimental import pallas as pl


def kernel(x, emb_weight):
    raise NotImplementedError("write your pallas kernel here")


# speedup vs baseline: 3.4490x; 3.4490x over previous
import jax
import jax.numpy as jnp
from jax import lax
from jax.experimental import pallas as pl
from jax.experimental.pallas import tpu as pltpu
from jax.experimental.pallas import tpu_sc as plsc

D = 64
IDX_ROW = 128
CHUNK = 512
SUB = CHUNK // IDX_ROW


def _relu_body(t_ref, o_ref):
    o_ref[...] = jnp.maximum(t_ref[...], 0.0)


def _sc_body(b_per_w, nc):
    n_chunks = b_per_w // CHUNK

    def body(idx_hbm, table_hbm, out_hbm, idx_v, rows0, rows1, gsem0, gsem1):
        wid = lax.axis_index("s") * nc + lax.axis_index("c")
        row_base = wid * b_per_w
        idx_row_base = wid * (b_per_w // IDX_ROW)
        pltpu.sync_copy(
            idx_hbm.at[pl.ds(idx_row_base, b_per_w // IDX_ROW)], idx_v
        )

        def fire(c, rows, sem):
            for j in range(SUB):
                pltpu.async_copy(
                    table_hbm.at[idx_v.at[c * SUB + j]],
                    rows.at[pl.ds(j * IDX_ROW, IDX_ROW)],
                    sem,
                )

        def drain(rows, sem):
            pltpu.make_async_copy(
                out_hbm.at[pl.ds(row_base, CHUNK)], rows, sem
            ).wait()

        def out_copy(c, rows):
            pltpu.sync_copy(rows, out_hbm.at[pl.ds(row_base + c * CHUNK, CHUNK)])

        fire(0, rows0, gsem0)

        def pair_body(p, _):
            c0 = 2 * p
            fire(c0 + 1, rows1, gsem1)
            drain(rows0, gsem0)
            out_copy(c0, rows0)

            @pl.when(p < n_chunks // 2 - 1)
            def _():
                fire(c0 + 2, rows0, gsem0)

            drain(rows1, gsem1)
            out_copy(c0 + 1, rows1)
            return 0

        lax.fori_loop(0, n_chunks // 2, pair_body, 0)

    return body


def kernel(x, emb_weight):
    b, h = x.shape
    vocab, d = emb_weight.shape
    assert d == D
    n_idx = b * h

    relu_table = pl.pallas_call(
        _relu_body,
        out_shape=jax.ShapeDtypeStruct((vocab, d), jnp.float32),
    )(emb_weight)

    info = plsc.get_sparse_core_info()
    nw = info.num_cores * info.num_subcores
    b_per_w = n_idx // nw
    assert b_per_w * nw == n_idx and b_per_w % (2 * CHUNK) == 0

    idx2d = x.reshape(-1, IDX_ROW)
    mesh = plsc.VectorSubcoreMesh(core_axis_name="c", subcore_axis_name="s")
    out = pl.kernel(
        _sc_body(b_per_w, info.num_cores),
        out_type=jax.ShapeDtypeStruct((n_idx, D), jnp.float32),
        mesh=mesh,
        compiler_params=pltpu.CompilerParams(
            needs_layout_passes=False, use_tc_tiling_on_sc=False
        ),
        scratch_types=[
            pltpu.VMEM((b_per_w // IDX_ROW, IDX_ROW), jnp.int32),
            pltpu.VMEM((CHUNK, D), jnp.float32),
            pltpu.VMEM((CHUNK, D), jnp.float32),
            pltpu.SemaphoreType.DMA,
            pltpu.SemaphoreType.DMA,
        ],
    )(idx2d, relu_table)
    return out.reshape(b, h, D)


# speedup vs baseline: 4.9707x; 1.4412x over previous
import jax
import jax.numpy as jnp
from jax import lax
from jax.experimental import pallas as pl
from jax.experimental.pallas import tpu as pltpu
from jax.experimental.pallas import tpu_sc as plsc

D = 64
IDX_ROW = 100
CHUNK_B = 2
RELU_TILES = 8


def _sc_body(vocab, hist, batches_per_w, nc):
    relu_rows = vocab // RELU_TILES
    n_chunks = batches_per_w // CHUNK_B
    sub_per_chunk = CHUNK_B * hist // IDX_ROW

    def body(idx_hbm, table_hbm, out_hbm, shared_tab, tab_v, idx_v, rows0,
             rows1, gsem0, gsem1):
        cid = lax.axis_index("c")
        sid = lax.axis_index("s")
        wid = sid * nc + cid

        @pl.when(sid < RELU_TILES)
        def _():
            r0 = sid * relu_rows
            pltpu.sync_copy(table_hbm.at[pl.ds(r0, relu_rows)], tab_v)

            def relu_row(r, _):
                for k in range(D // 16):
                    v = tab_v[r, pl.ds(k * 16, 16)]
                    tab_v[r, pl.ds(k * 16, 16)] = jnp.maximum(v, 0.0)
                return 0

            lax.fori_loop(0, relu_rows, relu_row, 0)
            pltpu.sync_copy(tab_v, shared_tab.at[pl.ds(r0, relu_rows)])

        plsc.subcore_barrier()

        batch_base = wid * batches_per_w
        idx_row_base = wid * (batches_per_w * hist // IDX_ROW)
        pltpu.sync_copy(
            idx_hbm.at[pl.ds(idx_row_base, n_chunks * sub_per_chunk)], idx_v
        )

        def fire(c, rows, sem):
            for j in range(sub_per_chunk):
                pltpu.async_copy(
                    shared_tab.at[idx_v.at[c * sub_per_chunk + j]],
                    rows.at[j // 2, pl.ds((j % 2) * IDX_ROW, IDX_ROW)],
                    sem,
                )

        def drain(rows, sem):
            pltpu.make_async_copy(
                out_hbm.at[pl.ds(batch_base, CHUNK_B)], rows, sem
            ).wait()

        def out_copy(c, rows):
            pltpu.sync_copy(
                rows, out_hbm.at[pl.ds(batch_base + c * CHUNK_B, CHUNK_B)]
            )

        fire(0, rows0, gsem0)

        def pair_body(p, _):
            c0 = 2 * p
            fire(c0 + 1, rows1, gsem1)
            drain(rows0, gsem0)
            out_copy(c0, rows0)

            @pl.when(p < n_chunks // 2 - 1)
            def _():
                fire(c0 + 2, rows0, gsem0)

            drain(rows1, gsem1)
            out_copy(c0 + 1, rows1)
            return 0

        lax.fori_loop(0, n_chunks // 2, pair_body, 0)

    return body


def kernel(x, emb_weight):
    b, h = x.shape
    vocab, d = emb_weight.shape
    assert d == D and h % IDX_ROW == 0

    info = plsc.get_sparse_core_info()
    nw = info.num_cores * info.num_subcores
    batches_per_w = b // nw
    assert batches_per_w * nw == b and batches_per_w % (2 * CHUNK_B) == 0

    idx2d = x.reshape(-1, IDX_ROW)
    mesh = plsc.VectorSubcoreMesh(core_axis_name="c", subcore_axis_name="s")
    out = pl.kernel(
        _sc_body(vocab, h, batches_per_w, info.num_cores),
        out_type=jax.ShapeDtypeStruct((b, h, D), jnp.float32),
        mesh=mesh,
        compiler_params=pltpu.CompilerParams(
            needs_layout_passes=False, use_tc_tiling_on_sc=False
        ),
        scratch_types=[
            pltpu.VMEM_SHARED((vocab, D), jnp.float32),
            pltpu.VMEM((vocab // RELU_TILES, D), jnp.float32),
            pltpu.VMEM((batches_per_w * h // IDX_ROW, IDX_ROW), jnp.int32),
            pltpu.VMEM((CHUNK_B, h, D), jnp.float32),
            pltpu.VMEM((CHUNK_B, h, D), jnp.float32),
            pltpu.SemaphoreType.DMA,
            pltpu.SemaphoreType.DMA,
        ],
    )(idx2d, emb_weight)
    return out


# speedup vs baseline: 8.0991x; 1.6294x over previous
import jax
import jax.numpy as jnp
from jax import lax
from jax.experimental import pallas as pl
from jax.experimental.pallas import tpu as pltpu
from jax.experimental.pallas import tpu_sc as plsc

D = 64
IDX_ROW = 100
CHUNK_B = 1
RELU_TILES = 8


def _sc_body(vocab, hist, batches_per_w, nc):
    relu_rows = vocab // RELU_TILES
    n_chunks = batches_per_w // CHUNK_B
    sub_per_chunk = CHUNK_B * hist // IDX_ROW

    def body(idx_hbm, table_hbm, out_hbm, shared_tab, tab_v, idx_v, rows0,
             rows1, gsem0, gsem1):
        cid = lax.axis_index("c")
        sid = lax.axis_index("s")
        wid = sid * nc + cid

        @pl.when(sid < RELU_TILES)
        def _():
            r0 = sid * relu_rows
            pltpu.sync_copy(table_hbm.at[pl.ds(r0, relu_rows)], tab_v)

            def relu_row(r, _):
                for k in range(2 * D // 16):
                    v = tab_v[r, pl.ds(k * 16, 16)]
                    tab_v[r, pl.ds(k * 16, 16)] = jnp.maximum(v, 0.0)
                return 0

            lax.fori_loop(0, relu_rows, relu_row, 0)
            pltpu.sync_copy(tab_v, shared_tab.at[pl.ds(r0, relu_rows)])

        plsc.subcore_barrier()

        batch_base = wid * batches_per_w
        idx_row_base = wid * (batches_per_w * hist // IDX_ROW)
        pltpu.sync_copy(
            idx_hbm.at[pl.ds(idx_row_base, n_chunks * sub_per_chunk)], idx_v
        )

        def fire(c, rows, sem):
            for j in range(sub_per_chunk):
                pltpu.async_copy(
                    shared_tab.at[idx_v.at[c * sub_per_chunk + j]],
                    rows.at[pl.ds(j * IDX_ROW, IDX_ROW)],
                    sem,
                )

        def drain(rows, sem):
            pltpu.make_async_copy(
                out_hbm.at[pl.ds(batch_base * hist, CHUNK_B * hist)], rows, sem
            ).wait()

        def out_copy(c, rows):
            pltpu.sync_copy(
                rows,
                out_hbm.at[
                    pl.ds((batch_base + c * CHUNK_B) * hist, CHUNK_B * hist)
                ],
            )

        fire(0, rows0, gsem0)

        def pair_body(p, _):
            c0 = 2 * p
            fire(c0 + 1, rows1, gsem1)
            drain(rows0, gsem0)
            out_copy(c0, rows0)

            @pl.when(p < n_chunks // 2 - 1)
            def _():
                fire(c0 + 2, rows0, gsem0)

            drain(rows1, gsem1)
            out_copy(c0 + 1, rows1)
            return 0

        lax.fori_loop(0, n_chunks // 2, pair_body, 0)

    return body


def kernel(x, emb_weight):
    b, h = x.shape
    vocab, d = emb_weight.shape
    assert d == D and h % IDX_ROW == 0

    info = plsc.get_sparse_core_info()
    nw = info.num_cores * info.num_subcores
    batches_per_w = b // nw
    assert batches_per_w * nw == b and batches_per_w % (2 * CHUNK_B) == 0

    idx2d = x.reshape(-1, IDX_ROW)
    table_pad = jnp.pad(emb_weight, ((0, 0), (0, D)))
    mesh = plsc.VectorSubcoreMesh(core_axis_name="c", subcore_axis_name="s")
    out = pl.kernel(
        _sc_body(vocab, h, batches_per_w, info.num_cores),
        out_type=jax.ShapeDtypeStruct((b * h, 2 * D), jnp.float32),
        mesh=mesh,
        compiler_params=pltpu.CompilerParams(
            needs_layout_passes=False, use_tc_tiling_on_sc=False
        ),
        scratch_types=[
            pltpu.VMEM_SHARED((vocab, 2 * D), jnp.float32),
            pltpu.VMEM((vocab // RELU_TILES, 2 * D), jnp.float32),
            pltpu.VMEM((batches_per_w * h // IDX_ROW, IDX_ROW), jnp.int32),
            pltpu.VMEM((CHUNK_B * h, 2 * D), jnp.float32),
            pltpu.VMEM((CHUNK_B * h, 2 * D), jnp.float32),
            pltpu.SemaphoreType.DMA,
            pltpu.SemaphoreType.DMA,
        ],
    )(idx2d, table_pad)
    return out[:, :D].reshape(b, h, D)


# speedup vs baseline: 8.4632x; 1.0449x over previous
import jax
import jax.numpy as jnp
from jax import lax
from jax.experimental import pallas as pl
from jax.experimental.pallas import tpu as pltpu
from jax.experimental.pallas import tpu_sc as plsc

D = 64
IDX_ROW = 100
CHUNK_B = 1
RELU_TILES = 8


def _sc_body(vocab, hist, batches_per_w, nc):
    relu_rows = vocab // RELU_TILES
    n_chunks = batches_per_w // CHUNK_B
    sub_per_chunk = CHUNK_B * hist // IDX_ROW

    def body(idx_hbm, table_hbm, out_hbm, shared_tab, tab_v, idx_v, rows0,
             rows1, gsem0, gsem1):
        cid = lax.axis_index("c")
        sid = lax.axis_index("s")
        wid = sid * nc + cid

        @pl.when(sid < RELU_TILES)
        def _():
            r0 = sid * relu_rows
            pltpu.sync_copy(table_hbm.at[pl.ds(r0, relu_rows)], tab_v)

            def relu_row(r, _):
                for k in range(2 * D // 16):
                    v = tab_v[r, pl.ds(k * 16, 16)]
                    tab_v[r, pl.ds(k * 16, 16)] = jnp.maximum(v, 0.0)
                return 0

            lax.fori_loop(0, relu_rows, relu_row, 0)
            pltpu.sync_copy(tab_v, shared_tab.at[pl.ds(r0, relu_rows)])

        plsc.subcore_barrier()

        batch_base = wid * batches_per_w
        idx_row_base = wid * (batches_per_w * hist // IDX_ROW)
        pltpu.sync_copy(
            idx_hbm.at[pl.ds(idx_row_base, n_chunks * sub_per_chunk)], idx_v
        )

        def fire(c, rows, sem):
            for j in range(sub_per_chunk):
                pltpu.async_copy(
                    shared_tab.at[idx_v.at[c * sub_per_chunk + j]],
                    rows.at[pl.ds(j * IDX_ROW, IDX_ROW)],
                    sem,
                )

        def drain(rows, sem):
            pltpu.make_async_copy(
                out_hbm.at[pl.ds(batch_base * hist, CHUNK_B * hist)], rows, sem
            ).wait()

        def out_copy(c, rows):
            pltpu.sync_copy(
                rows.at[:, pl.ds(0, D)],
                out_hbm.at[
                    pl.ds((batch_base + c * CHUNK_B) * hist, CHUNK_B * hist),
                    pl.ds(0, D),
                ],
            )

        fire(0, rows0, gsem0)

        def pair_body(p, _):
            c0 = 2 * p
            fire(c0 + 1, rows1, gsem1)
            drain(rows0, gsem0)
            out_copy(c0, rows0)

            @pl.when(p < n_chunks // 2 - 1)
            def _():
                fire(c0 + 2, rows0, gsem0)

            drain(rows1, gsem1)
            out_copy(c0 + 1, rows1)
            return 0

        lax.fori_loop(0, n_chunks // 2, pair_body, 0)

    return body


def kernel(x, emb_weight):
    b, h = x.shape
    vocab, d = emb_weight.shape
    assert d == D and h % IDX_ROW == 0

    info = plsc.get_sparse_core_info()
    nw = info.num_cores * info.num_subcores
    batches_per_w = b // nw
    assert batches_per_w * nw == b and batches_per_w % (2 * CHUNK_B) == 0

    idx2d = x.reshape(-1, IDX_ROW)
    table_pad = jnp.pad(emb_weight, ((0, 0), (0, D)))
    mesh = plsc.VectorSubcoreMesh(core_axis_name="c", subcore_axis_name="s")
    out = pl.kernel(
        _sc_body(vocab, h, batches_per_w, info.num_cores),
        out_type=jax.ShapeDtypeStruct((b * h, 2 * D), jnp.float32),
        mesh=mesh,
        compiler_params=pltpu.CompilerParams(
            needs_layout_passes=False, use_tc_tiling_on_sc=False
        ),
        scratch_types=[
            pltpu.VMEM_SHARED((vocab, 2 * D), jnp.float32),
            pltpu.VMEM((vocab // RELU_TILES, 2 * D), jnp.float32),
            pltpu.VMEM((batches_per_w * h // IDX_ROW, IDX_ROW), jnp.int32),
            pltpu.VMEM((CHUNK_B * h, 2 * D), jnp.float32),
            pltpu.VMEM((CHUNK_B * h, 2 * D), jnp.float32),
            pltpu.SemaphoreType.DMA,
            pltpu.SemaphoreType.DMA,
        ],
    )(idx2d, table_pad)
    return out[:, :D].reshape(b, h, D)


# speedup vs baseline: 10.2040x; 1.2057x over previous
import jax
import jax.numpy as jnp
from jax import lax
from jax.experimental import pallas as pl
from jax.experimental.pallas import tpu as pltpu
from jax.experimental.pallas import tpu_sc as plsc

D = 64
IDX_ROW = 100
CHUNK_B = 1
RELU_TILES = 8


def _sc_body(vocab, hist, batches_per_w, nc):
    relu_rows = vocab // RELU_TILES
    n_chunks = batches_per_w // CHUNK_B
    sub_per_chunk = CHUNK_B * hist // IDX_ROW

    def body(idx_hbm, table_hbm, out_hbm, shared_tab, tab_v, idx_v, rows0,
             rows1, gsem0, gsem1):
        cid = lax.axis_index("c")
        sid = lax.axis_index("s")
        wid = sid * nc + cid

        @pl.when(sid < RELU_TILES)
        def _():
            r0 = sid * relu_rows
            pltpu.sync_copy(table_hbm.at[pl.ds(r0, relu_rows)], tab_v)

            def relu_row(r, _):
                for k in range(D // 16):
                    v = tab_v[r, pl.ds(k * 16, 16)]
                    tab_v[r, pl.ds(k * 16, 16)] = jnp.maximum(v, 0.0)
                return 0

            lax.fori_loop(0, relu_rows, relu_row, 0)
            pltpu.sync_copy(tab_v, shared_tab.at[pl.ds(r0, relu_rows)])

        plsc.subcore_barrier()

        batch_base = wid * batches_per_w
        idx_row_base = wid * (batches_per_w * hist // IDX_ROW)
        pltpu.sync_copy(
            idx_hbm.at[pl.ds(idx_row_base, n_chunks * sub_per_chunk)], idx_v
        )

        def fire(c, rows, sem):
            for j in range(sub_per_chunk):
                pltpu.async_copy(
                    shared_tab.at[idx_v.at[c * sub_per_chunk + j]],
                    rows.at[pl.ds(j * IDX_ROW, IDX_ROW)],
                    sem,
                )

        def drain(rows, sem):
            pltpu.make_async_copy(
                out_hbm.at[pl.ds(batch_base * hist, CHUNK_B * hist)], rows, sem
            ).wait()

        def out_copy(c, rows):
            pltpu.sync_copy(
                rows,
                out_hbm.at[
                    pl.ds((batch_base + c * CHUNK_B) * hist, CHUNK_B * hist),
                    pl.ds(0, D),
                ],
            )

        fire(0, rows0, gsem0)

        def pair_body(p, _):
            c0 = 2 * p
            fire(c0 + 1, rows1, gsem1)
            drain(rows0, gsem0)
            out_copy(c0, rows0)

            @pl.when(p < n_chunks // 2 - 1)
            def _():
                fire(c0 + 2, rows0, gsem0)

            drain(rows1, gsem1)
            out_copy(c0 + 1, rows1)
            return 0

        lax.fori_loop(0, n_chunks // 2, pair_body, 0)

    return body


def kernel(x, emb_weight):
    b, h = x.shape
    vocab, d = emb_weight.shape
    assert d == D and h % IDX_ROW == 0

    info = plsc.get_sparse_core_info()
    nw = info.num_cores * info.num_subcores
    batches_per_w = b // nw
    assert batches_per_w * nw == b and batches_per_w % (2 * CHUNK_B) == 0

    idx2d = x.reshape(-1, IDX_ROW)
    mesh = plsc.VectorSubcoreMesh(core_axis_name="c", subcore_axis_name="s")
    out = pl.kernel(
        _sc_body(vocab, h, batches_per_w, info.num_cores),
        out_type=jax.ShapeDtypeStruct((b * h, 2 * D), jnp.float32),
        mesh=mesh,
        compiler_params=pltpu.CompilerParams(
            needs_layout_passes=False, use_tc_tiling_on_sc=False
        ),
        scratch_types=[
            pltpu.VMEM_SHARED((vocab, D), jnp.float32),
            pltpu.VMEM((vocab // RELU_TILES, D), jnp.float32),
            pltpu.VMEM((batches_per_w * h // IDX_ROW, IDX_ROW), jnp.int32),
            pltpu.VMEM((CHUNK_B * h, D), jnp.float32),
            pltpu.VMEM((CHUNK_B * h, D), jnp.float32),
            pltpu.SemaphoreType.DMA,
            pltpu.SemaphoreType.DMA,
        ],
    )(idx2d, emb_weight)
    return out[:, :D].reshape(b, h, D)


# speedup vs baseline: 10.2253x; 1.0021x over previous
import jax
import jax.numpy as jnp
from jax import lax
from jax.experimental import pallas as pl
from jax.experimental.pallas import tpu as pltpu
from jax.experimental.pallas import tpu_sc as plsc

D = 64
IDX_ROW = 100
CHUNK_B = 2
RELU_TILES = 8


def _sc_body(vocab, hist, batches_per_w, nc):
    relu_rows = vocab // RELU_TILES
    n_chunks = batches_per_w // CHUNK_B
    sub_per_chunk = CHUNK_B * hist // IDX_ROW

    def body(idx_hbm, table_hbm, out_hbm, shared_tab, tab_v, idx_v, rows0,
             rows1, gsem0, gsem1):
        cid = lax.axis_index("c")
        sid = lax.axis_index("s")
        wid = sid * nc + cid

        @pl.when(sid < RELU_TILES)
        def _():
            r0 = sid * relu_rows
            pltpu.sync_copy(table_hbm.at[pl.ds(r0, relu_rows)], tab_v)

            def relu_row(r, _):
                for k in range(D // 16):
                    v = tab_v[r, pl.ds(k * 16, 16)]
                    tab_v[r, pl.ds(k * 16, 16)] = jnp.maximum(v, 0.0)
                return 0

            lax.fori_loop(0, relu_rows, relu_row, 0)
            pltpu.sync_copy(tab_v, shared_tab.at[pl.ds(r0, relu_rows)])

        plsc.subcore_barrier()

        batch_base = wid * batches_per_w
        idx_row_base = wid * (batches_per_w * hist // IDX_ROW)
        pltpu.sync_copy(
            idx_hbm.at[pl.ds(idx_row_base, n_chunks * sub_per_chunk)], idx_v
        )

        def fire(c, rows, sem):
            for j in range(sub_per_chunk):
                pltpu.async_copy(
                    shared_tab.at[idx_v.at[c * sub_per_chunk + j]],
                    rows.at[pl.ds(j * IDX_ROW, IDX_ROW)],
                    sem,
                )

        def drain(rows, sem):
            pltpu.make_async_copy(
                out_hbm.at[pl.ds(batch_base * hist, CHUNK_B * hist)], rows, sem
            ).wait()

        def out_copy(c, rows):
            pltpu.sync_copy(
                rows,
                out_hbm.at[
                    pl.ds((batch_base + c * CHUNK_B) * hist, CHUNK_B * hist),
                    pl.ds(0, D),
                ],
            )

        fire(0, rows0, gsem0)

        def pair_body(p, _):
            c0 = 2 * p
            fire(c0 + 1, rows1, gsem1)
            drain(rows0, gsem0)
            out_copy(c0, rows0)

            @pl.when(p < n_chunks // 2 - 1)
            def _():
                fire(c0 + 2, rows0, gsem0)

            drain(rows1, gsem1)
            out_copy(c0 + 1, rows1)
            return 0

        lax.fori_loop(0, n_chunks // 2, pair_body, 0)

    return body


def kernel(x, emb_weight):
    b, h = x.shape
    vocab, d = emb_weight.shape
    assert d == D and h % IDX_ROW == 0

    info = plsc.get_sparse_core_info()
    nw = info.num_cores * info.num_subcores
    batches_per_w = b // nw
    assert batches_per_w * nw == b and batches_per_w % (2 * CHUNK_B) == 0

    idx2d = x.reshape(-1, IDX_ROW)
    mesh = plsc.VectorSubcoreMesh(core_axis_name="c", subcore_axis_name="s")
    out = pl.kernel(
        _sc_body(vocab, h, batches_per_w, info.num_cores),
        out_type=jax.ShapeDtypeStruct((b * h, 2 * D), jnp.float32),
        mesh=mesh,
        compiler_params=pltpu.CompilerParams(
            needs_layout_passes=False, use_tc_tiling_on_sc=False
        ),
        scratch_types=[
            pltpu.VMEM_SHARED((vocab, D), jnp.float32),
            pltpu.VMEM((vocab // RELU_TILES, D), jnp.float32),
            pltpu.VMEM((batches_per_w * h // IDX_ROW, IDX_ROW), jnp.int32),
            pltpu.VMEM((CHUNK_B * h, D), jnp.float32),
            pltpu.VMEM((CHUNK_B * h, D), jnp.float32),
            pltpu.SemaphoreType.DMA,
            pltpu.SemaphoreType.DMA,
        ],
    )(idx2d, emb_weight)
    return out[:, :D].reshape(b, h, D)


# speedup vs baseline: 10.2254x; 1.0000x over previous
import jax
import jax.numpy as jnp
from jax import lax
from jax.experimental import pallas as pl
from jax.experimental.pallas import tpu as pltpu
from jax.experimental.pallas import tpu_sc as plsc

D = 64
IDX_ROW = 100
CHUNK_B = 2
RELU_TILES = 8


def _sc_body(vocab, hist, batches_per_w, nc):
    relu_rows = vocab // RELU_TILES
    n_chunks = batches_per_w // CHUNK_B
    sub_per_chunk = CHUNK_B * hist // IDX_ROW

    def body(idx_hbm, table_hbm, out_hbm, shared_tab, tab_v, idx_v, rows0,
             rows1, gsem0, gsem1):
        cid = lax.axis_index("c")
        sid = lax.axis_index("s")
        wid = sid * nc + cid

        @pl.when(sid < RELU_TILES)
        def _():
            r0 = sid * relu_rows
            pltpu.sync_copy(table_hbm.at[pl.ds(r0, relu_rows)], tab_v)

            def relu_row(r, _):
                for k in range(D // 16):
                    v = tab_v[r, pl.ds(k * 16, 16)]
                    tab_v[r, pl.ds(k * 16, 16)] = jnp.maximum(v, 0.0)
                return 0

            lax.fori_loop(0, relu_rows, relu_row, 0)
            pltpu.sync_copy(tab_v, shared_tab.at[pl.ds(r0, relu_rows)])

        plsc.subcore_barrier()

        batch_base = wid * batches_per_w
        idx_row_base = wid * (batches_per_w * hist // IDX_ROW)
        pltpu.sync_copy(
            idx_hbm.at[pl.ds(idx_row_base, n_chunks * sub_per_chunk)], idx_v
        )

        def fire(c, rows, sem):
            for j in range(sub_per_chunk):
                pltpu.async_copy(
                    shared_tab.at[idx_v.at[c * sub_per_chunk + j]],
                    rows.at[pl.ds(j * IDX_ROW, IDX_ROW)],
                    sem,
                )

        def drain(rows, sem):
            pltpu.make_async_copy(
                out_hbm.at[
                    pl.ds(batch_base * hist, CHUNK_B * hist), pl.ds(0, D)
                ],
                rows,
                sem,
            ).wait()

        def out_copy(c, rows):
            pltpu.sync_copy(
                rows,
                out_hbm.at[
                    pl.ds((batch_base + c * CHUNK_B) * hist, CHUNK_B * hist),
                    pl.ds(0, D),
                ],
            )

        fire(0, rows0, gsem0)

        def pair_body(p, _):
            c0 = 2 * p
            fire(c0 + 1, rows1, gsem1)
            drain(rows0, gsem0)
            out_copy(c0, rows0)

            @pl.when(p < n_chunks // 2 - 1)
            def _():
                fire(c0 + 2, rows0, gsem0)

            drain(rows1, gsem1)
            out_copy(c0 + 1, rows1)
            return 0

        lax.fori_loop(0, n_chunks // 2, pair_body, 0)

    return body


def kernel(x, emb_weight):
    b, h = x.shape
    vocab, d = emb_weight.shape
    assert d == D and h % IDX_ROW == 0

    info = plsc.get_sparse_core_info()
    nw = info.num_cores * info.num_subcores
    batches_per_w = b // nw
    assert batches_per_w * nw == b and batches_per_w % (2 * CHUNK_B) == 0

    idx2d = x.reshape(-1, IDX_ROW)
    mesh = plsc.VectorSubcoreMesh(core_axis_name="c", subcore_axis_name="s")
    out = pl.kernel(
        _sc_body(vocab, h, batches_per_w, info.num_cores),
        out_type=jax.ShapeDtypeStruct((b * h, 2 * D), jnp.float32),
        mesh=mesh,
        compiler_params=pltpu.CompilerParams(
            needs_layout_passes=False, use_tc_tiling_on_sc=False
        ),
        scratch_types=[
            pltpu.VMEM_SHARED((vocab, D), jnp.float32),
            pltpu.VMEM((vocab // RELU_TILES, D), jnp.float32),
            pltpu.VMEM((batches_per_w * h // IDX_ROW, IDX_ROW), jnp.int32),
            pltpu.VMEM((CHUNK_B * h, D), jnp.float32),
            pltpu.VMEM((CHUNK_B * h, D), jnp.float32),
            pltpu.SemaphoreType.DMA,
            pltpu.SemaphoreType.DMA,
        ],
    )(idx2d, emb_weight)
    return out[:, :D].reshape(b, h, D)
